# baseline (speedup 1.0000x reference)
import functools

import jax
import jax.numpy as jnp
from jax import lax
from jax.experimental import pallas as pl
from jax.experimental.pallas import tpu as pltpu
from jax.experimental.pallas import tpu_sc as plsc

_B, _K, _L = 64, 21, 27
_V = [98, 195, 389, 778]
_C = [256, 128, 64, 32]
_VP = [104, 200, 392, 784]


_BT = 8


def _front_body(lat_ref, px_ref, py_ref, wde_ref, bde_ref, u_ref, out_ref):
    for bi in range(_BT):
        lat = lat_ref[bi]
        g = jnp.dot(wde_ref[...], lat,
                    preferred_element_type=jnp.float32) + bde_ref[...]
        px = px_ref[bi] * 7.0
        py = py_ref[bi] * 7.0
        x0f = jnp.floor(px)
        y0f = jnp.floor(py)
        wx = px - x0f
        wy = py - y0f
        x0 = jnp.clip(x0f, 0.0, 7.0).astype(jnp.int32)
        x1 = jnp.clip(x0f + 1.0, 0.0, 7.0).astype(jnp.int32)
        y0 = jnp.clip(y0f, 0.0, 7.0).astype(jnp.int32)
        y1 = jnp.clip(y0f + 1.0, 0.0, 7.0).astype(jnp.int32)
        iota_p = lax.broadcasted_iota(jnp.int32, (64, _K), 0)

        def oh(yi, xi, w):
            return jnp.where(iota_p == yi * 8 + xi, w, 0.0)

        wb = (oh(y0, x0, (1.0 - wx) * (1.0 - wy)) + oh(y0, x1, wx * (1.0 - wy))
              + oh(y1, x0, (1.0 - wx) * wy) + oh(y1, x1, wx * wy))
        uw = lax.dot_general(u_ref[...], wb, (((1,), (1,)), ((), ())),
                             preferred_element_type=jnp.float32)
        x0b = lax.dot_general(uw, g, (((1,), (1,)), ((), ())),
                              preferred_element_type=jnp.float32)
        out_ref[:, bi, :] = x0b


def _front(lat3, px, py, wde, bde2, upad):
    return pl.pallas_call(
        _front_body,
        grid=(_B // _BT,),
        in_specs=[
            pl.BlockSpec((_BT, 256, 64), lambda b: (b, 0, 0)),
            pl.BlockSpec((_BT, 1, _K), lambda b: (b, 0, 0)),
            pl.BlockSpec((_BT, 1, _K), lambda b: (b, 0, 0)),
            pl.BlockSpec((256, 256), lambda b: (0, 0)),
            pl.BlockSpec((256, 1), lambda b: (0, 0)),
            pl.BlockSpec((_VP[0], _K), lambda b: (0, 0)),
        ],
        out_specs=pl.BlockSpec((_VP[0], _BT, 256), lambda b: (0, b, 0)),
        out_shape=jax.ShapeDtypeStruct((_VP[0], _B, 256), jnp.float32),
    )(lat3, px, py, wde, bde2, upad)


def _mm_body(a_ref, b_ref, o_ref):
    o_ref[...] = jnp.dot(a_ref[...], b_ref[...],
                         preferred_element_type=jnp.float32)


def _matmul_cols(s, x, n_tile):
    m, kd = s.shape
    n = x.shape[1]
    return pl.pallas_call(
        _mm_body,
        grid=(n // n_tile,),
        in_specs=[
            pl.BlockSpec((m, kd), lambda i: (0, 0)),
            pl.BlockSpec((kd, n_tile), lambda i: (0, i)),
        ],
        out_specs=pl.BlockSpec((m, n_tile), lambda i: (0, i)),
        out_shape=jax.ShapeDtypeStruct((m, n), jnp.float32),
    )(s, x)


def _expand_body(vt, rw, rz, y_ref, w_ref, o_ref):
    acc = jnp.dot(y_ref[...], w_ref[...],
                  preferred_element_type=jnp.float32)
    for j in range(_L):
        blk = acc[:, j * 128:(j + 1) * 128].reshape(vt, rw, 128)
        if rz == rw:
            o_ref[j] = blk
        else:
            o_ref[j, :, 0:rw, :] = blk


def _expand(y2, wbig, vp, vt, rw, rz):
    kdim = y2.shape[1]
    body = functools.partial(_expand_body, vt, rw, rz)
    return pl.pallas_call(
        body,
        grid=(vp // vt,),
        in_specs=[
            pl.BlockSpec((vt * rw, kdim), lambda m: (m, 0)),
            pl.BlockSpec((kdim, _L * 128), lambda m: (0, 0)),
        ],
        out_specs=pl.BlockSpec((_L, vt, rz, 128), lambda m: (0, m, 0, 0)),
        out_shape=jax.ShapeDtypeStruct((_L, vp, rz, 128), jnp.float32),
    )(y2, wbig)


def _gather_reduce(z3, idx32, btile, vp, v_real, segs, rz, rread, relu):
    wd = rread * 128
    items = vp * segs
    ipw = -(-items // 32)
    mesh = plsc.VectorSubcoreMesh(core_axis_name="c", subcore_axis_name="s")

    @functools.partial(
        pl.kernel,
        out_type=jax.ShapeDtypeStruct((32, ipw, wd), jnp.float32),
        mesh=mesh,
        scratch_types=[
            pltpu.VMEM((ipw, 32), jnp.int32),
            pltpu.VMEM((_L, rz, 128), jnp.float32),
            pltpu.VMEM((_L, rz, 128), jnp.float32),
            pltpu.VMEM((wd,), jnp.float32),
            pltpu.VMEM((wd,), jnp.float32),
            pltpu.SemaphoreType.DMA,
            pltpu.SemaphoreType.DMA,
        ],
    )
    def k(z_hbm, idx_hbm, b_hbm, out_hbm, idx_v, gb0, gb1, acc, bias_v,
          sem0, sem1):
        wid = lax.axis_index("s") * 2 + lax.axis_index("c")
        t0 = wid * ipw
        pltpu.sync_copy(idx_hbm.at[wid], idx_v)
        pltpu.sync_copy(b_hbm, bias_v)
        gbufs = (gb0, gb1)
        sems = (sem0, sem1)

        def start(slot, r):
            pltpu.make_async_copy(
                z_hbm.at[idx_v.at[r, pl.ds(0, _L)]], gbufs[slot],
                sems[slot]).start()

        def finish(slot, r):
            pltpu.make_async_copy(
                z_hbm.at[idx_v.at[r, pl.ds(0, _L)]], gbufs[slot],
                sems[slot]).wait()

        def process(slot, r):
            gbuf = gbufs[slot]
            t = t0 + r
            valid = t < v_real * segs

            def row(b, cb):
                def chunk(oc, cc):
                    dsg = pl.ds(oc * 16, 16)
                    s = gbuf[0, b, dsg]
                    for i in range(1, _L):
                        s = s + gbuf[i, b, dsg]
                    ds = pl.ds(b * 128 + oc * 16, 16)
                    s = s + bias_v[ds]
                    if relu:
                        s = jnp.maximum(s, 0.0)
                    acc[ds] = jnp.where(valid, s, 0.0)
                    return cc

                lax.fori_loop(0, 8, chunk, 0)
                return cb

            lax.fori_loop(0, rread, row, 0)

            @pl.when(t < items)
            def _():
                pltpu.sync_copy(acc, out_hbm.at[wid, r])

        start(0, 0)

        def step(rr, carry):
            r = 2 * rr

            @pl.when(r + 1 < ipw)
            def _():
                start(1, r + 1)

            finish(0, r)
            process(0, r)

            @pl.when(r + 2 < ipw)
            def _():
                start(0, r + 2)

            @pl.when(r + 1 < ipw)
            def _():
                finish(1, r + 1)
                process(1, r + 1)

            return carry

        lax.fori_loop(0, -(-ipw // 2), step, 0)

    return k(z3, idx32, btile)


def _build_idx(sidx, v_real, vp, segs):
    j = jnp.arange(_L, dtype=jnp.int32)[None, :]
    base = (sidx.astype(jnp.int32) + j * vp) * segs
    base = base[:, None, :] + jnp.arange(segs, dtype=jnp.int32)[None, :, None]
    base = base.reshape(v_real * segs, _L)
    zrow = jnp.int32(v_real * segs)
    items = vp * segs
    ipw = -(-items // 32)
    m = jnp.full((ipw * 32, 32), zrow, jnp.int32)
    return m.at[:v_real * segs, :_L].set(base).reshape(32, ipw, 32)


def _build_s(rows, cols, vals, vout_p, vin_p):
    pr = (rows[:, None] == jnp.arange(vout_p, dtype=rows.dtype)[None, :])
    pc = (cols[:, None] == jnp.arange(vin_p, dtype=cols.dtype)[None, :])
    prw = pr.astype(jnp.float32) * vals[:, None]
    return jnp.dot(prw.T, pc.astype(jnp.float32),
                   preferred_element_type=jnp.float32)


def _build_wbig(w, cin, cout):
    q = 128 // cout
    w3 = w.reshape(_L, cin, cout).transpose(1, 0, 2)
    eye = jnp.eye(q, dtype=jnp.float32)
    big = (eye[:, None, None, :, None]
           * w3[None, :, :, None, :])
    return big.reshape(q * cin, _L * 128)


def kernel(pred2d_pt, latent, W_de, b_de, U, W0, b0, W1, b1, W2, b2, Wh, bh,
           sidx195, sidx389, sidx778,
           rows0, cols0, vals0, rows1, cols1, vals1, rows2, cols2, vals2):
    lat3 = latent.reshape(_B, 256, 64)
    px = pred2d_pt[:, :, 0].reshape(_B, 1, _K)
    py = pred2d_pt[:, :, 1].reshape(_B, 1, _K)
    bde2 = b_de.reshape(256, 1)
    upad = jnp.zeros((_VP[0], _K), jnp.float32).at[:_V[0]].set(U)

    s_mats = [
        _build_s(rows0, cols0, vals0, _VP[1], _VP[0]),
        _build_s(rows1, cols1, vals1, _VP[2], _VP[1]),
        _build_s(rows2, cols2, vals2, _VP[3], _VP[2]),
    ]
    w_bigs = [
        _build_wbig(W0, _C[0], _C[1]),
        _build_wbig(W1, _C[1], _C[2]),
        _build_wbig(W2, _C[2], _C[3]),
    ]
    wh8 = jnp.zeros((_L, _C[3], 8), jnp.float32).at[:, :, :3].set(
        Wh.reshape(_L, _C[3], 3))
    wh_big = _build_wbig(wh8.reshape(_L * _C[3], 8), _C[3], 8)
    bh8 = jnp.zeros((8,), jnp.float32).at[:3].set(bh)

    hb = _B // 2
    seg_l = [2, 1, 1]
    rz_l = [16, 16, 8]
    vts = [8, 14, 28]
    idxs = [
        _build_idx(sidx195, _V[1], _VP[1], seg_l[0]),
        _build_idx(sidx389, _V[2], _VP[2], seg_l[1]),
        _build_idx(sidx778, _V[3], _VP[3], seg_l[2]),
    ]
    seg_h = 1
    idx_h = _build_idx(sidx778, _V[3], _VP[3], seg_h)
    btiles = [jnp.tile(b0, hb // seg_l[0]), jnp.tile(b1, hb // seg_l[1]),
              jnp.tile(b2, hb // seg_l[2])]
    btile_h = jnp.tile(bh8, _B // seg_h)

    x = _front(lat3, px, py, W_de, bde2, upad)
    halves = [x[:, :hb, :].reshape(_VP[0], hb * _C[0]),
              x[:, hb:, :].reshape(_VP[0], hb * _C[0])]

    for i in range(3):
        cin, cout = _C[i], _C[i + 1]
        q = 128 // cout
        vp = _VP[i + 1]
        segs = seg_l[i]
        rw = hb // q
        rslab = rz_l[i]
        zs = []
        for h in range(2):
            y = _matmul_cols(s_mats[i], halves[h], 2048)
            z4 = _expand(y.reshape(vp * hb // q, q * cin), w_bigs[i],
                         vp, vts[i], rw, rw)
            zs.append(z4.reshape(_L * vp * segs, rslab, 128))
        for h in range(2):
            xo = _gather_reduce(zs[h], idxs[i], btiles[i], vp, _V[i + 1],
                                segs, rslab, rslab, relu=True)
            wd = rslab * 128
            halves[h] = (xo.reshape(-1, wd)[:vp * segs]
                         .reshape(vp, hb * cout))

    x = jnp.concatenate(halves, axis=1)
    zh = _expand(x.reshape(_VP[3] * _B // 16, 16 * _C[3]), wh_big,
                 _VP[3], 56, 4, 8)
    zh3 = zh.reshape(_L * _VP[3], 8, 128)
    xh = _gather_reduce(zh3, idx_h, btile_h, _VP[3], _V[3],
                        seg_h, 8, 4, relu=False)
    xh = xh.reshape(-1, _B * 8)[:_VP[3]]
    verts = xh.reshape(_VP[3], _B, 8)[:_V[3], :, :3].transpose(1, 0, 2)
    return (verts, pred2d_pt)


# speedup vs baseline: 1.1494x; 1.1494x over previous
import functools

import jax
import jax.numpy as jnp
from jax import lax
from jax.experimental import pallas as pl
from jax.experimental.pallas import tpu as pltpu
from jax.experimental.pallas import tpu_sc as plsc

_B, _K, _L = 64, 21, 27
_V = [98, 195, 389, 778]
_C = [256, 128, 64, 32]
_VP = [104, 200, 392, 784]


_BT = 8


def _front_body(lat_ref, px_ref, py_ref, wde_ref, bde_ref, u_ref, out_ref):
    for bi in range(_BT):
        lat = lat_ref[bi]
        g = jnp.dot(wde_ref[...], lat,
                    preferred_element_type=jnp.float32) + bde_ref[...]
        px = px_ref[bi] * 7.0
        py = py_ref[bi] * 7.0
        x0f = jnp.floor(px)
        y0f = jnp.floor(py)
        wx = px - x0f
        wy = py - y0f
        x0 = jnp.clip(x0f, 0.0, 7.0).astype(jnp.int32)
        x1 = jnp.clip(x0f + 1.0, 0.0, 7.0).astype(jnp.int32)
        y0 = jnp.clip(y0f, 0.0, 7.0).astype(jnp.int32)
        y1 = jnp.clip(y0f + 1.0, 0.0, 7.0).astype(jnp.int32)
        iota_p = lax.broadcasted_iota(jnp.int32, (64, _K), 0)

        def oh(yi, xi, w):
            return jnp.where(iota_p == yi * 8 + xi, w, 0.0)

        wb = (oh(y0, x0, (1.0 - wx) * (1.0 - wy)) + oh(y0, x1, wx * (1.0 - wy))
              + oh(y1, x0, (1.0 - wx) * wy) + oh(y1, x1, wx * wy))
        uw = lax.dot_general(u_ref[...], wb, (((1,), (1,)), ((), ())),
                             preferred_element_type=jnp.float32)
        x0b = lax.dot_general(uw, g, (((1,), (1,)), ((), ())),
                              preferred_element_type=jnp.float32)
        out_ref[:, bi, :] = x0b


def _front(lat3, px, py, wde, bde2, upad):
    return pl.pallas_call(
        _front_body,
        grid=(_B // _BT,),
        in_specs=[
            pl.BlockSpec((_BT, 256, 64), lambda b: (b, 0, 0)),
            pl.BlockSpec((_BT, 1, _K), lambda b: (b, 0, 0)),
            pl.BlockSpec((_BT, 1, _K), lambda b: (b, 0, 0)),
            pl.BlockSpec((256, 256), lambda b: (0, 0)),
            pl.BlockSpec((256, 1), lambda b: (0, 0)),
            pl.BlockSpec((_VP[0], _K), lambda b: (0, 0)),
        ],
        out_specs=pl.BlockSpec((_VP[0], _BT, 256), lambda b: (0, b, 0)),
        out_shape=jax.ShapeDtypeStruct((_VP[0], _B, 256), jnp.float32),
    )(lat3, px, py, wde, bde2, upad)


def _mm_body(a_ref, b_ref, o_ref):
    o_ref[...] = jnp.dot(a_ref[...], b_ref[...],
                         preferred_element_type=jnp.float32)


def _matmul_cols(s, x, n_tile):
    m, kd = s.shape
    n = x.shape[1]
    return pl.pallas_call(
        _mm_body,
        grid=(n // n_tile,),
        in_specs=[
            pl.BlockSpec((m, kd), lambda i: (0, 0)),
            pl.BlockSpec((kd, n_tile), lambda i: (0, i)),
        ],
        out_specs=pl.BlockSpec((m, n_tile), lambda i: (0, i)),
        out_shape=jax.ShapeDtypeStruct((m, n), jnp.float32),
    )(s, x)


def _expand_body(vt, rw, rz, y_ref, w_ref, o_ref):
    acc = jnp.dot(y_ref[...], w_ref[...],
                  preferred_element_type=jnp.float32)
    for j in range(_L):
        blk = acc[:, j * 128:(j + 1) * 128].reshape(vt, rw, 128)
        if rz == rw:
            o_ref[j] = blk
        else:
            o_ref[j, :, 0:rw, :] = blk


def _expand(y2, wbig, vp, vt, rw, rz):
    kdim = y2.shape[1]
    body = functools.partial(_expand_body, vt, rw, rz)
    return pl.pallas_call(
        body,
        grid=(vp // vt,),
        in_specs=[
            pl.BlockSpec((vt * rw, kdim), lambda m: (m, 0)),
            pl.BlockSpec((kdim, _L * 128), lambda m: (0, 0)),
        ],
        out_specs=pl.BlockSpec((_L, vt, rz, 128), lambda m: (0, m, 0, 0)),
        out_shape=jax.ShapeDtypeStruct((_L, vp, rz, 128), jnp.float32),
    )(y2, wbig)


def _gather_reduce(z3, idx32, btile, vp, v_real, segs, rz, rread, relu):
    wd = rread * 128
    items = vp * segs
    ipw = -(-items // 32)
    mesh = plsc.VectorSubcoreMesh(core_axis_name="c", subcore_axis_name="s")

    @functools.partial(
        pl.kernel,
        out_type=jax.ShapeDtypeStruct((32, ipw, wd), jnp.float32),
        mesh=mesh,
        scratch_types=[
            pltpu.VMEM((ipw, 32), jnp.int32),
            pltpu.VMEM((_L, rz, 128), jnp.float32),
            pltpu.VMEM((_L, rz, 128), jnp.float32),
            pltpu.VMEM((wd,), jnp.float32),
            pltpu.VMEM((wd,), jnp.float32),
            pltpu.SemaphoreType.DMA,
            pltpu.SemaphoreType.DMA,
        ],
    )
    def k(z_hbm, idx_hbm, b_hbm, out_hbm, idx_v, gb0, gb1, acc, bias_v,
          sem0, sem1):
        wid = lax.axis_index("s") * 2 + lax.axis_index("c")
        t0 = wid * ipw
        pltpu.sync_copy(idx_hbm.at[wid], idx_v)
        pltpu.sync_copy(b_hbm, bias_v)
        gbufs = (gb0, gb1)
        sems = (sem0, sem1)

        def start(slot, r):
            pltpu.make_async_copy(
                z_hbm.at[idx_v.at[r, pl.ds(0, _L)]], gbufs[slot],
                sems[slot]).start()

        def finish(slot, r):
            pltpu.make_async_copy(
                z_hbm.at[idx_v.at[r, pl.ds(0, _L)]], gbufs[slot],
                sems[slot]).wait()

        def process(slot, r):
            gbuf = gbufs[slot]
            t = t0 + r
            valid = t < v_real * segs

            def row(b, cb):
                for oc in range(8):
                    dsg = pl.ds(oc * 16, 16)
                    s = gbuf[0, b, dsg]
                    for i in range(1, _L):
                        s = s + gbuf[i, b, dsg]
                    ds = pl.ds(b * 128 + oc * 16, 16)
                    s = s + bias_v[ds]
                    if relu:
                        s = jnp.maximum(s, 0.0)
                    acc[ds] = jnp.where(valid, s, 0.0)
                return cb

            lax.fori_loop(0, rread, row, 0)

            @pl.when(t < items)
            def _():
                pltpu.sync_copy(acc, out_hbm.at[wid, r])

        start(0, 0)

        def step(rr, carry):
            r = 2 * rr

            @pl.when(r + 1 < ipw)
            def _():
                start(1, r + 1)

            finish(0, r)
            process(0, r)

            @pl.when(r + 2 < ipw)
            def _():
                start(0, r + 2)

            @pl.when(r + 1 < ipw)
            def _():
                finish(1, r + 1)
                process(1, r + 1)

            return carry

        lax.fori_loop(0, -(-ipw // 2), step, 0)

    return k(z3, idx32, btile)


def _build_idx(sidx, v_real, vp, segs):
    j = jnp.arange(_L, dtype=jnp.int32)[None, :]
    base = (sidx.astype(jnp.int32) + j * vp) * segs
    base = base[:, None, :] + jnp.arange(segs, dtype=jnp.int32)[None, :, None]
    base = base.reshape(v_real * segs, _L)
    zrow = jnp.int32(v_real * segs)
    items = vp * segs
    ipw = -(-items // 32)
    m = jnp.full((ipw * 32, 32), zrow, jnp.int32)
    return m.at[:v_real * segs, :_L].set(base).reshape(32, ipw, 32)


def _build_s(rows, cols, vals, vout_p, vin_p):
    pr = (rows[:, None] == jnp.arange(vout_p, dtype=rows.dtype)[None, :])
    pc = (cols[:, None] == jnp.arange(vin_p, dtype=cols.dtype)[None, :])
    prw = pr.astype(jnp.float32) * vals[:, None]
    return jnp.dot(prw.T, pc.astype(jnp.float32),
                   preferred_element_type=jnp.float32)


def _build_wbig(w, cin, cout):
    q = 128 // cout
    w3 = w.reshape(_L, cin, cout).transpose(1, 0, 2)
    eye = jnp.eye(q, dtype=jnp.float32)
    big = (eye[:, None, None, :, None]
           * w3[None, :, :, None, :])
    return big.reshape(q * cin, _L * 128)


def kernel(pred2d_pt, latent, W_de, b_de, U, W0, b0, W1, b1, W2, b2, Wh, bh,
           sidx195, sidx389, sidx778,
           rows0, cols0, vals0, rows1, cols1, vals1, rows2, cols2, vals2):
    lat3 = latent.reshape(_B, 256, 64)
    px = pred2d_pt[:, :, 0].reshape(_B, 1, _K)
    py = pred2d_pt[:, :, 1].reshape(_B, 1, _K)
    bde2 = b_de.reshape(256, 1)
    upad = jnp.zeros((_VP[0], _K), jnp.float32).at[:_V[0]].set(U)

    s_mats = [
        _build_s(rows0, cols0, vals0, _VP[1], _VP[0]),
        _build_s(rows1, cols1, vals1, _VP[2], _VP[1]),
        _build_s(rows2, cols2, vals2, _VP[3], _VP[2]),
    ]
    w_bigs = [
        _build_wbig(W0, _C[0], _C[1]),
        _build_wbig(W1, _C[1], _C[2]),
        _build_wbig(W2, _C[2], _C[3]),
    ]
    wh8 = jnp.zeros((_L, _C[3], 8), jnp.float32).at[:, :, :3].set(
        Wh.reshape(_L, _C[3], 3))
    wh_big = _build_wbig(wh8.reshape(_L * _C[3], 8), _C[3], 8)
    bh8 = jnp.zeros((8,), jnp.float32).at[:3].set(bh)

    seg_l = [4, 2, 1]
    seg_h = 1
    idxs = [
        _build_idx(sidx195, _V[1], _VP[1], seg_l[0]),
        _build_idx(sidx389, _V[2], _VP[2], seg_l[1]),
        _build_idx(sidx778, _V[3], _VP[3], seg_l[2]),
    ]
    idx_h = _build_idx(sidx778, _V[3], _VP[3], seg_h)
    btiles = [jnp.tile(b0, _B // seg_l[0]), jnp.tile(b1, _B // seg_l[1]),
              jnp.tile(b2, _B // seg_l[2])]
    btile_h = jnp.tile(bh8, _B // seg_h)

    x = _front(lat3, px, py, W_de, bde2, upad)
    x = x.reshape(_VP[0], _B * _C[0])

    vts = [8, 14, 28]
    for i in range(3):
        cin, cout = _C[i], _C[i + 1]
        q = 128 // cout
        vp = _VP[i + 1]
        segs = seg_l[i]
        rw = _B // q
        rslab = rw // segs
        y = _matmul_cols(s_mats[i], x, 2048)
        z4 = _expand(y.reshape(vp * _B // q, q * cin), w_bigs[i],
                     vp, vts[i], rw, rw)
        z3 = z4.reshape(_L * vp * segs, rslab, 128)
        x = _gather_reduce(z3, idxs[i], btiles[i], vp, _V[i + 1],
                           segs, rslab, rslab, relu=True)
        wd = (_B // segs) * cout
        x = x.reshape(-1, wd)[:vp * segs].reshape(vp, _B * cout)

    zh = _expand(x.reshape(_VP[3] * _B // 16, 16 * _C[3]), wh_big,
                 _VP[3], 56, 4, 8)
    zh3 = zh.reshape(_L * _VP[3], 8, 128)
    xh = _gather_reduce(zh3, idx_h, btile_h, _VP[3], _V[3],
                        seg_h, 8, 4, relu=False)
    xh = xh.reshape(-1, _B * 8)[:_VP[3]]
    verts = xh.reshape(_VP[3], _B, 8)[:_V[3], :, :3].transpose(1, 0, 2)
    return (verts, pred2d_pt)
